# baseline (speedup 1.0000x reference)
import functools

import jax
import jax.numpy as jnp
from jax import lax
from jax.experimental import pallas as pl
from jax.experimental.pallas import tpu as pltpu
from jax.experimental.pallas import tpu_sc as plsc

N, RANK, CSUB, CFULL = 4096, 8, 16, 384
PSEUDO = 10.0

NS, L = 16, 16
SPW = N // NS
ACC = RANK * CFULL
RED = ACC // NS
WROW = NS * ACC


@functools.partial(
    pl.kernel,
    out_type=jax.ShapeDtypeStruct((ACC,), jnp.float32),
    mesh=plsc.VectorSubcoreMesh(core_axis_name="c", subcore_axis_name="s",
                                num_cores=1),
    compiler_params=pltpu.CompilerParams(needs_layout_passes=False),
    scratch_types=[
        pltpu.VMEM((SPW * RANK * CSUB,), jnp.float32),
        pltpu.VMEM((SPW * CSUB,), jnp.int32),
        pltpu.VMEM((SPW,), jnp.float32),
        pltpu.VMEM((ACC,), jnp.float32),
        pltpu.VMEM((RED,), jnp.float32),
        pltpu.VMEM((RED,), jnp.float32),
        pltpu.VMEM((RED,), jnp.float32),
        pltpu.VMEM((NS * L,), jnp.float32),
        pltpu.VMEM((L,), jnp.float32),
        pltpu.VMEM_SHARED((NS * ACC + NS * L,), jnp.float32),
    ],
)
def _sc_fused(feat_hbm, ch_hbm, w_hbm, nm_hbm, out_hbm,
              feat_v, ch_v, w_v, acc_v, red_v, tmp_v, nm_v, wall_v, wst_v, shared):
    sid = lax.axis_index("s")
    base = sid * SPW

    pltpu.sync_copy(feat_hbm.at[pl.ds(base * RANK * CSUB, SPW * RANK * CSUB)], feat_v)
    pltpu.sync_copy(ch_hbm.at[pl.ds(base * CSUB, SPW * CSUB)], ch_v)
    pltpu.sync_copy(w_hbm.at[pl.ds(base, SPW)], w_v)

    def zero_body(i, _):
        acc_v[pl.ds(pl.multiple_of(i * L, L), L)] = jnp.zeros((L,), jnp.float32)
        return 0
    lax.fori_loop(0, ACC // L, zero_body, 0)

    iota = lax.iota(jnp.int32, L)
    shift = jnp.minimum(iota + 1, L - 1)
    last_lane = iota == L - 1

    @plsc.parallel_loop(0, SPW, unroll=4)
    def spike_body(n):
        ch = ch_v[pl.ds(pl.multiple_of(n * CSUB, CSUB), CSUB)]
        ch_next = ch.at[shift].get(mode="promise_in_bounds")
        keep = (ch != ch_next) | last_lane
        wg = w_v[pl.ds(pl.multiple_of((n // L) * L, L), L)]
        w = wg.at[jnp.full((L,), n % L, jnp.int32)].get(mode="promise_in_bounds")
        idx = ch
        for r in range(RANK):
            row = feat_v[pl.ds(pl.multiple_of((n * RANK + r) * CSUB, CSUB), CSUB)]
            plsc.addupdate_scatter(acc_v, [idx], row * w, mask=keep)
            if r < RANK - 1:
                idx = idx + CFULL

    wsum = w_v[pl.ds(0, L)]
    for g in range(1, SPW // L):
        wsum = wsum + w_v[pl.ds(g * L, L)]
    wst_v[...] = wsum

    pltpu.sync_copy(acc_v, shared.at[pl.ds(pl.multiple_of(sid * ACC, ACC), ACC)])
    pltpu.sync_copy(wst_v, shared.at[pl.ds(WROW + pl.multiple_of(sid * L, L), L)])
    plsc.subcore_barrier()

    pltpu.sync_copy(shared.at[pl.ds(WROW, NS * L)], wall_v)
    wtot_vec = wall_v[pl.ds(0, L)]
    for s in range(1, NS):
        wtot_vec = wtot_vec + wall_v[pl.ds(s * L, L)]
    cs = plsc.cumsum(wtot_vec)
    total_b = cs.at[jnp.full((L,), L - 1, jnp.int32)].get(mode="promise_in_bounds")
    scale = 1.0 / (PSEUDO + total_b)

    off = pl.multiple_of(sid * RED, RED)
    pltpu.sync_copy(shared.at[pl.ds(off, RED)], red_v)

    def red_body(s, _):
        pltpu.sync_copy(shared.at[pl.ds(pl.multiple_of(s * ACC, ACC) + off, RED)], tmp_v)
        for k in range(RED // L):
            sl = pl.ds(k * L, L)
            red_v[sl] = red_v[sl] + tmp_v[sl]
        return 0
    lax.fori_loop(1, NS, red_body, 0)

    pltpu.sync_copy(nm_hbm.at[pl.ds(off, RED)], nm_v)
    for k in range(RED // L):
        sl = pl.ds(k * L, L)
        red_v[sl] = (red_v[sl] + PSEUDO * nm_v[sl]) * scale

    pltpu.sync_copy(red_v, out_hbm.at[pl.ds(off, RED)])


def kernel(features, channels, weights, noise_mean_full):
    out = _sc_fused(features.reshape(-1),
                    channels.astype(jnp.int32).reshape(-1),
                    weights,
                    noise_mean_full.reshape(-1))
    return out.reshape(RANK, CFULL)


# speedup vs baseline: 1.6095x; 1.6095x over previous
import functools

import jax
import jax.numpy as jnp
from jax import lax
from jax.experimental import pallas as pl
from jax.experimental.pallas import tpu as pltpu
from jax.experimental.pallas import tpu_sc as plsc

N, RANK, CSUB, CFULL = 4096, 8, 16, 384
PSEUDO = 10.0

NC, NS, L = 2, 16, 16
NW = NC * NS
SPW = N // NW
NB = SPW // L
ACC = RANK * CFULL
RED = ACC // NS


@functools.partial(
    pl.kernel,
    out_type=jax.ShapeDtypeStruct((NC * ACC,), jnp.float32),
    mesh=plsc.VectorSubcoreMesh(core_axis_name="c", subcore_axis_name="s"),
    compiler_params=pltpu.CompilerParams(needs_layout_passes=False),
    scratch_types=[
        pltpu.VMEM((RANK * CSUB, SPW), jnp.float32),
        pltpu.VMEM((CSUB, SPW), jnp.int32),
        pltpu.VMEM((SPW,), jnp.float32),
        pltpu.VMEM((ACC,), jnp.float32),
        pltpu.VMEM((RED,), jnp.float32),
        pltpu.VMEM((RED,), jnp.float32),
        pltpu.VMEM_SHARED((NS * ACC,), jnp.float32),
    ],
)
def _sc_partials(feat_hbm, ch_hbm, w_hbm, out_hbm,
                 feat_v, ch_v, w_v, acc_v, red_v, tmp_v, shared):
    cid = lax.axis_index("c")
    sid = lax.axis_index("s")
    wid = cid * NS + sid
    base = wid * SPW

    pltpu.sync_copy(feat_hbm.at[:, pl.ds(base, SPW)], feat_v)
    pltpu.sync_copy(ch_hbm.at[:, pl.ds(base, SPW)], ch_v)
    pltpu.sync_copy(w_hbm.at[pl.ds(base, SPW)], w_v)

    def zero_body(i, _):
        acc_v[pl.ds(pl.multiple_of(i * L, L), L)] = jnp.zeros((L,), jnp.float32)
        return 0
    lax.fori_loop(0, ACC // L, zero_body, 0)

    @plsc.parallel_loop(0, NB, unroll=1)
    def block_body(b):
        n0 = pl.multiple_of(b * L, L)
        wv = w_v[pl.ds(n0, L)]
        ch_cur = ch_v[0, pl.ds(n0, L)]
        for j in range(CSUB):
            if j < CSUB - 1:
                ch_nxt = ch_v[j + 1, pl.ds(n0, L)]
                keep = ch_cur != ch_nxt
            else:
                ch_nxt = ch_cur
                keep = None
            idx = ch_cur
            for r in range(RANK):
                row = feat_v[r * CSUB + j, pl.ds(n0, L)]
                plsc.addupdate_scatter(acc_v, [idx], row * wv, mask=keep)
                if r < RANK - 1:
                    idx = idx + CFULL
            ch_cur = ch_nxt

    pltpu.sync_copy(acc_v, shared.at[pl.ds(pl.multiple_of(sid * ACC, ACC), ACC)])
    plsc.subcore_barrier()

    off = pl.multiple_of(sid * RED, RED)
    pltpu.sync_copy(shared.at[pl.ds(off, RED)], red_v)

    def red_body(s, _):
        pltpu.sync_copy(shared.at[pl.ds(pl.multiple_of(s * ACC, ACC) + off, RED)], tmp_v)
        for k in range(RED // L):
            sl = pl.ds(k * L, L)
            red_v[sl] = red_v[sl] + tmp_v[sl]
        return 0
    lax.fori_loop(1, NS, red_body, 0)

    pltpu.sync_copy(red_v, out_hbm.at[pl.ds(cid * ACC + off, RED)])


def _finish_body(p_ref, w_ref, nm_ref, o_ref):
    total_w = jnp.sum(w_ref[...])
    s = p_ref[0] + p_ref[1]
    o_ref[...] = (s + PSEUDO * nm_ref[...]) * (1.0 / (PSEUDO + total_w))


def kernel(features, channels, weights, noise_mean_full):
    feat_t = features.transpose(1, 2, 0).reshape(RANK * CSUB, N)
    ch_t = channels.astype(jnp.int32).transpose(1, 0)
    partials = _sc_partials(feat_t, ch_t, weights)
    partials = partials.reshape(NC, RANK, CFULL)
    out = pl.pallas_call(
        _finish_body,
        out_shape=jax.ShapeDtypeStruct((RANK, CFULL), jnp.float32),
    )(partials, weights.reshape(NS * NC, SPW), noise_mean_full)
    return out


# speedup vs baseline: 1.7083x; 1.0614x over previous
import functools

import jax
import jax.numpy as jnp
from jax import lax
from jax.experimental import pallas as pl
from jax.experimental.pallas import tpu as pltpu
from jax.experimental.pallas import tpu_sc as plsc

N, RANK, CSUB, CFULL = 4096, 8, 16, 384
PSEUDO = 10.0

NC, NS, L = 2, 16, 16
NW = NC * NS
SPW = N // NW
NB = SPW // L
ACC = RANK * CFULL
RED = ACC // NS


@functools.partial(
    pl.kernel,
    out_type=jax.ShapeDtypeStruct((NC * ACC,), jnp.float32),
    mesh=plsc.VectorSubcoreMesh(core_axis_name="c", subcore_axis_name="s"),
    compiler_params=pltpu.CompilerParams(needs_layout_passes=False),
    scratch_types=[
        pltpu.VMEM((RANK * CSUB, SPW), jnp.float32),
        pltpu.VMEM((CSUB, SPW), jnp.int32),
        pltpu.VMEM((SPW,), jnp.float32),
        pltpu.VMEM((ACC,), jnp.float32),
        pltpu.VMEM((RED,), jnp.float32),
        pltpu.VMEM((NS * RED,), jnp.float32),
        pltpu.VMEM_SHARED((NS * ACC,), jnp.float32),
        pltpu.SemaphoreType.DMA,
        pltpu.SemaphoreType.DMA,
    ],
)
def _sc_partials(feat_hbm, ch_hbm, w_hbm, out_hbm,
                 feat_v, ch_v, w_v, acc_v, red_v, stage_v, shared,
                 sem_in, sem_red):
    cid = lax.axis_index("c")
    sid = lax.axis_index("s")
    wid = cid * NS + sid
    base = wid * SPW
    off = pl.multiple_of(sid * RED, RED)

    cp_f = pltpu.async_copy(feat_hbm.at[:, pl.ds(base, SPW)], feat_v, sem_in)
    cp_c = pltpu.async_copy(ch_hbm.at[:, pl.ds(base, SPW)], ch_v, sem_in)
    cp_w = pltpu.async_copy(w_hbm.at[pl.ds(base, SPW)], w_v, sem_in)

    def zero_body(i, _):
        acc_v[pl.ds(pl.multiple_of(i * L, L), L)] = jnp.zeros((L,), jnp.float32)
        return 0
    lax.fori_loop(0, ACC // L, zero_body, 0)

    cp_f.wait()
    cp_c.wait()
    cp_w.wait()

    @plsc.parallel_loop(0, NB, unroll=2)
    def block_body(b):
        n0 = pl.multiple_of(b * L, L)
        wv = w_v[pl.ds(n0, L)]
        ch_cur = ch_v[0, pl.ds(n0, L)]
        for j in range(CSUB):
            if j < CSUB - 1:
                ch_nxt = ch_v[j + 1, pl.ds(n0, L)]
                keep = ch_cur != ch_nxt
            else:
                ch_nxt = ch_cur
                keep = None
            for r in range(RANK):
                row = feat_v[r * CSUB + j, pl.ds(n0, L)]
                plsc.addupdate_scatter(acc_v, [ch_cur + r * CFULL],
                                       row * wv, mask=keep)
            ch_cur = ch_nxt

    pltpu.sync_copy(acc_v, shared.at[pl.ds(pl.multiple_of(sid * ACC, ACC), ACC)])
    plsc.subcore_barrier()

    cps = []
    for s in range(NS):
        src = shared.at[pl.ds(pl.multiple_of(s * ACC, ACC) + off, RED)]
        dst = stage_v.at[pl.ds(s * RED, RED)]
        cps.append(pltpu.async_copy(src, dst, sem_red))
    for cp in cps:
        cp.wait()

    for k in range(RED // L):
        sl = pl.ds(k * L, L)
        red_v[sl] = stage_v[sl]
    for s in range(1, NS):
        b0 = s * RED
        for k in range(RED // L):
            sl = pl.ds(k * L, L)
            red_v[sl] = red_v[sl] + stage_v[pl.ds(b0 + k * L, L)]

    pltpu.sync_copy(red_v, out_hbm.at[pl.ds(cid * ACC + off, RED)])


def _finish_body(p_ref, w_ref, nm_ref, o_ref):
    total_w = jnp.sum(w_ref[...])
    s = p_ref[0] + p_ref[1]
    o_ref[...] = (s + PSEUDO * nm_ref[...]) * (1.0 / (PSEUDO + total_w))


def kernel(features, channels, weights, noise_mean_full):
    feat_t = features.transpose(1, 2, 0).reshape(RANK * CSUB, N)
    ch_t = channels.astype(jnp.int32).transpose(1, 0)
    partials = _sc_partials(feat_t, ch_t, weights)
    partials = partials.reshape(NC, RANK, CFULL)
    out = pl.pallas_call(
        _finish_body,
        out_shape=jax.ShapeDtypeStruct((RANK, CFULL), jnp.float32),
    )(partials, weights.reshape(NS * NC, SPW), noise_mean_full)
    return out


# speedup vs baseline: 1.7571x; 1.0285x over previous
import functools

import jax
import jax.numpy as jnp
from jax import lax
from jax.experimental import pallas as pl
from jax.experimental.pallas import tpu as pltpu
from jax.experimental.pallas import tpu_sc as plsc

N, RANK, CSUB, CFULL = 4096, 8, 16, 384
PSEUDO = 10.0

NC, NS, L = 2, 16, 16
NW = NC * NS
SPW = N // NW
NB = SPW // L
ACC = RANK * CFULL


@functools.partial(
    pl.kernel,
    out_type=jax.ShapeDtypeStruct((NW * ACC,), jnp.float32),
    mesh=plsc.VectorSubcoreMesh(core_axis_name="c", subcore_axis_name="s"),
    compiler_params=pltpu.CompilerParams(needs_layout_passes=False),
    scratch_types=[
        pltpu.VMEM((RANK * CSUB, SPW), jnp.float32),
        pltpu.VMEM((CSUB, SPW), jnp.int32),
        pltpu.VMEM((SPW,), jnp.float32),
        pltpu.VMEM((ACC,), jnp.float32),
        pltpu.SemaphoreType.DMA,
    ],
)
def _sc_partials(feat_hbm, ch_hbm, w_hbm, out_hbm,
                 feat_v, ch_v, w_v, acc_v, sem_in):
    cid = lax.axis_index("c")
    sid = lax.axis_index("s")
    wid = cid * NS + sid
    base = wid * SPW

    cp_f = pltpu.async_copy(feat_hbm.at[:, pl.ds(base, SPW)], feat_v, sem_in)
    cp_c = pltpu.async_copy(ch_hbm.at[:, pl.ds(base, SPW)], ch_v, sem_in)
    cp_w = pltpu.async_copy(w_hbm.at[pl.ds(base, SPW)], w_v, sem_in)

    def zero_body(i, _):
        acc_v[pl.ds(pl.multiple_of(i * L, L), L)] = jnp.zeros((L,), jnp.float32)
        return 0
    lax.fori_loop(0, ACC // L, zero_body, 0)

    cp_f.wait()
    cp_c.wait()
    cp_w.wait()

    @plsc.parallel_loop(0, NB, unroll=2)
    def block_body(b):
        n0 = pl.multiple_of(b * L, L)
        wv = w_v[pl.ds(n0, L)]
        ch_cur = ch_v[0, pl.ds(n0, L)]
        for j in range(CSUB):
            if j < CSUB - 1:
                ch_nxt = ch_v[j + 1, pl.ds(n0, L)]
                keep = ch_cur != ch_nxt
            else:
                ch_nxt = ch_cur
                keep = None
            for r in range(RANK):
                row = feat_v[r * CSUB + j, pl.ds(n0, L)]
                plsc.addupdate_scatter(acc_v, [ch_cur + r * CFULL],
                                       row * wv, mask=keep)
            ch_cur = ch_nxt

    pltpu.sync_copy(acc_v, out_hbm.at[pl.ds(wid * ACC, ACC)])


def _finish_body(p_ref, w_ref, nm_ref, o_ref):
    total_w = jnp.sum(w_ref[...])
    s = jnp.sum(p_ref[...], axis=0)
    o_ref[...] = (s + PSEUDO * nm_ref[...]) * (1.0 / (PSEUDO + total_w))


def kernel(features, channels, weights, noise_mean_full):
    feat_t = features.transpose(1, 2, 0).reshape(RANK * CSUB, N)
    ch_t = channels.astype(jnp.int32).transpose(1, 0)
    partials = _sc_partials(feat_t, ch_t, weights)
    p3 = partials.reshape(NW, ACC // 128, 128)
    nm24 = noise_mean_full.reshape(ACC // 128, 128)
    out24 = pl.pallas_call(
        _finish_body,
        out_shape=jax.ShapeDtypeStruct((ACC // 128, 128), jnp.float32),
    )(p3, weights.reshape(NW, SPW), nm24)
    return out24.reshape(RANK, CFULL)


# speedup vs baseline: 1.9066x; 1.0851x over previous
import functools

import jax
import jax.numpy as jnp
from jax import lax
from jax.experimental import pallas as pl
from jax.experimental.pallas import tpu as pltpu
from jax.experimental.pallas import tpu_sc as plsc

N, RANK, CSUB, CFULL = 4096, 8, 16, 384
PSEUDO = 10.0

NC, NS, L = 2, 16, 16
NW = NC * NS
SPW = N // NW
NB = SPW // L
ACC = RANK * CFULL


@functools.partial(
    pl.kernel,
    out_type=jax.ShapeDtypeStruct((NW * ACC,), jnp.float32),
    mesh=plsc.VectorSubcoreMesh(core_axis_name="c", subcore_axis_name="s"),
    compiler_params=pltpu.CompilerParams(needs_layout_passes=False),
    scratch_types=[
        pltpu.VMEM((RANK * CSUB, SPW), jnp.float32),
        pltpu.VMEM((CSUB, SPW), jnp.int32),
        pltpu.VMEM((SPW,), jnp.float32),
        pltpu.VMEM((ACC,), jnp.float32),
        pltpu.SemaphoreType.DMA,
    ],
)
def _sc_partials(feat_hbm, ch_hbm, w_hbm, out_hbm,
                 feat_v, ch_v, w_v, acc_v, sem_in):
    cid = lax.axis_index("c")
    sid = lax.axis_index("s")
    wid = cid * NS + sid
    base = wid * SPW

    cp_f = pltpu.async_copy(feat_hbm.at[:, pl.ds(base, SPW)], feat_v, sem_in)
    cp_c = pltpu.async_copy(ch_hbm.at[:, pl.ds(base, SPW)], ch_v, sem_in)
    cp_w = pltpu.async_copy(w_hbm.at[pl.ds(base, SPW)], w_v, sem_in)

    def zero_body(i, _):
        acc_v[pl.ds(pl.multiple_of(i * L, L), L)] = jnp.zeros((L,), jnp.float32)
        return 0
    lax.fori_loop(0, ACC // L, zero_body, 0)

    cp_f.wait()
    cp_c.wait()
    cp_w.wait()

    @plsc.parallel_loop(0, NB, unroll=2)
    def block_body(b):
        n0 = pl.multiple_of(b * L, L)
        wv = w_v[pl.ds(n0, L)]
        ch_cur = ch_v[0, pl.ds(n0, L)]
        for j in range(CSUB):
            if j < CSUB - 1:
                ch_nxt = ch_v[j + 1, pl.ds(n0, L)]
                keep = ch_cur != ch_nxt
            else:
                ch_nxt = ch_cur
                keep = None
            rows = [feat_v[r * CSUB + j, pl.ds(n0, L)] for r in range(RANK)]
            idxs = [ch_cur + r * CFULL for r in range(RANK)]
            vals = [row * wv for row in rows]
            for r in range(RANK):
                plsc.addupdate_scatter(acc_v, [idxs[r]], vals[r], mask=keep)
            ch_cur = ch_nxt

    pltpu.sync_copy(acc_v, out_hbm.at[pl.ds(wid * ACC, ACC)])


def _finish_body(p_ref, w_ref, nm_ref, o_ref):
    total_w = jnp.sum(w_ref[...])
    s = jnp.sum(p_ref[...], axis=0)
    o_ref[...] = (s + PSEUDO * nm_ref[...]) * (1.0 / (PSEUDO + total_w))


def kernel(features, channels, weights, noise_mean_full):
    feat_t = features.transpose(1, 2, 0).reshape(RANK * CSUB, N)
    ch_t = channels.astype(jnp.int32).transpose(1, 0)
    partials = _sc_partials(feat_t, ch_t, weights)
    p3 = partials.reshape(NW, ACC // 128, 128)
    nm24 = noise_mean_full.reshape(ACC // 128, 128)
    out24 = pl.pallas_call(
        _finish_body,
        out_shape=jax.ShapeDtypeStruct((ACC // 128, 128), jnp.float32),
    )(p3, weights.reshape(NW, SPW), nm24)
    return out24.reshape(RANK, CFULL)
